# baseline (device time: 24480 ns/iter reference)
import contextlib
import os

import jax
import jax.numpy as jnp
from jax import lax
from jax.experimental import pallas as pl
from jax.experimental.pallas import tpu as pltpu

_PROBE = os.environ.get("PROBE", "")
_DO_Y = _PROBE in ("", "y", "xy")
_DO_X = _PROBE in ("", "x", "xy")
_X_EAGER = _PROBE in ("x", "xy")

N_COLS = 512
HALF_M = 512

CHUNK_ROWS = (32, 64, 128, 128, 64, 48, 32, 16)
assert sum(CHUNK_ROWS) == HALF_M
N_CHUNK = len(CHUNK_ROWS)
OFFS = tuple(sum(CHUNK_ROWS[:i]) for i in range(N_CHUNK + 1))

STORE_GROUPS = ((0, 1, 2), (3,), (4, 5), (6, 7))
N_GROUP = len(STORE_GROUPS)


def kernel(x):
    _, m, n = x.shape

    def body(
        x_hbm, out_hbm, xloc, pbuf,
        y_send, y_recv, x_send, x_recv, loc_sems, out_sems,
    ):
        my_x = lax.axis_index("x")
        my_y = lax.axis_index("y")
        y_peer = (my_x, 1 - my_y)
        x_peer = (1 - my_x, my_y)

        row0 = my_x * HALF_M
        peer_row0 = (1 - my_x) * HALF_M
        my_cols = pl.ds(my_y * N_COLS, N_COLS)

        loc_a = pltpu.make_async_copy(
            x_hbm.at[0, pl.ds(row0, HALF_M), my_cols],
            xloc.at[:HALF_M, :],
            loc_sems.at[0],
        )
        loc_a.start()
        loc_b = pltpu.make_async_copy(
            x_hbm.at[0, pl.ds(peer_row0, HALF_M), my_cols],
            xloc.at[HALF_M:, :],
            loc_sems.at[1],
        )
        loc_b.start()

        barrier_sem = pltpu.get_barrier_semaphore()
        for nbr in (y_peer, x_peer):
            pl.semaphore_signal(
                barrier_sem, inc=1, device_id=nbr,
                device_id_type=pl.DeviceIdType.MESH,
            )
        pl.semaphore_wait(barrier_sem, 2)

        def fwd_rdma(c):
            return pltpu.make_async_remote_copy(
                src_ref=pbuf.at[OFFS[c]:OFFS[c + 1], :],
                dst_ref=pbuf.at[HALF_M + OFFS[c]:HALF_M + OFFS[c + 1], :],
                send_sem=x_send.at[c],
                recv_sem=x_recv.at[c],
                device_id=x_peer,
                device_id_type=pl.DeviceIdType.MESH,
            )

        y_rdmas = []
        if _DO_Y:
            for c in range(N_CHUNK):
                rdma = pltpu.make_async_remote_copy(
                    src_ref=x_hbm.at[
                        0,
                        pl.ds(row0 + OFFS[c], CHUNK_ROWS[c]),
                        pl.ds((1 - my_y) * N_COLS, N_COLS),
                    ],
                    dst_ref=pbuf.at[OFFS[c]:OFFS[c + 1], :],
                    send_sem=y_send.at[c],
                    recv_sem=y_recv.at[c],
                    device_id=y_peer,
                    device_id_type=pl.DeviceIdType.MESH,
                )
                rdma.start()
                y_rdmas.append(rdma)

        x_rdmas = []
        if _DO_X and _X_EAGER:
            for c in range(N_CHUNK):
                rdma = fwd_rdma(c)
                rdma.start()
                x_rdmas.append(rdma)

        loc_a.wait()

        out_dmas = []
        group_of_last_chunk = {g[-1]: gi for gi, g in enumerate(STORE_GROUPS)}
        for c in range(N_CHUNK):
            if _DO_Y:
                y_rdmas[c].wait_recv()
            if _DO_X and not _X_EAGER:
                rdma = fwd_rdma(c)
                rdma.start()
                x_rdmas.append(rdma)
            gi = group_of_last_chunk.get(c)
            if gi is not None:
                lo = OFFS[STORE_GROUPS[gi][0]]
                hi = OFFS[STORE_GROUPS[gi][-1] + 1]
                xloc[lo:hi, :] = xloc[lo:hi, :] + pbuf[lo:hi, :]
                dma = pltpu.make_async_copy(
                    xloc.at[lo:hi, :],
                    out_hbm.at[pl.ds(row0 + lo, hi - lo), :],
                    out_sems.at[gi],
                )
                dma.start()
                out_dmas.append(dma)

        loc_b.wait()

        for c in range(N_CHUNK):
            if _DO_X:
                fwd_rdma(c).wait_recv()
            gi = group_of_last_chunk.get(c)
            if gi is not None:
                glo = OFFS[STORE_GROUPS[gi][0]]
                ghi = OFFS[STORE_GROUPS[gi][-1] + 1]
                lo, hi = HALF_M + glo, HALF_M + ghi
                xloc[lo:hi, :] = xloc[lo:hi, :] + pbuf[lo:hi, :]
                dma = pltpu.make_async_copy(
                    xloc.at[lo:hi, :],
                    out_hbm.at[pl.ds(peer_row0 + glo, ghi - glo), :],
                    out_sems.at[N_GROUP + gi],
                )
                dma.start()
                out_dmas.append(dma)

        for dma in out_dmas:
            dma.wait()
        for rdma in y_rdmas + x_rdmas:
            rdma.wait_send()

    return pl.pallas_call(
        body,
        out_shape=jax.ShapeDtypeStruct((m, N_COLS), jnp.float32),
        in_specs=[pl.BlockSpec(memory_space=pltpu.MemorySpace.HBM)],
        out_specs=pl.BlockSpec(memory_space=pltpu.MemorySpace.HBM),
        scratch_shapes=[
            pltpu.VMEM((m, N_COLS), jnp.float32),
            pltpu.VMEM((m, N_COLS), jnp.float32),
            pltpu.SemaphoreType.DMA((N_CHUNK,)),
            pltpu.SemaphoreType.DMA((N_CHUNK,)),
            pltpu.SemaphoreType.DMA((N_CHUNK,)),
            pltpu.SemaphoreType.DMA((N_CHUNK,)),
            pltpu.SemaphoreType.DMA((2,)),
            pltpu.SemaphoreType.DMA((2 * N_GROUP,)),
        ],
        compiler_params=pltpu.CompilerParams(collective_id=0),
    )(x)


# device time: 22319 ns/iter; 1.0968x vs baseline; 1.0968x over previous
import os

import jax
import jax.numpy as jnp
from jax import lax
from jax.experimental import pallas as pl
from jax.experimental.pallas import tpu as pltpu

_PROBE = os.environ.get("PROBE", "")
_DO_Y = _PROBE in ("", "y", "xy")
_DO_X = _PROBE in ("", "x", "xy")
_X_EAGER = _PROBE in ("x", "xy")

N_COLS = 512
HALF_M = 512

CHUNK_ROWS = tuple(
    int(v) for v in os.environ.get("CHUNKS", ",".join(["32"] * 16)).split(",")
)
assert sum(CHUNK_ROWS) == HALF_M
N_CHUNK = len(CHUNK_ROWS)
OFFS = tuple(sum(CHUNK_ROWS[:i]) for i in range(N_CHUNK + 1))

STORE_GROUPS = []
_g, _rows = [], 0
for _c, _r in enumerate(CHUNK_ROWS):
    _g.append(_c)
    _rows += _r
    if _rows >= 128:
        STORE_GROUPS.append(tuple(_g))
        _g, _rows = [], 0
if _g:
    STORE_GROUPS.append(tuple(_g))
STORE_GROUPS = tuple(STORE_GROUPS)
N_GROUP = len(STORE_GROUPS)


def kernel(x):
    _, m, n = x.shape

    def body(
        x_hbm, out_hbm, xloc, pbuf,
        y_send, y_recv, x_send, x_recv, loc_sem, out_sems,
    ):
        my_x = lax.axis_index("x")
        my_y = lax.axis_index("y")
        y_peer = (my_x, 1 - my_y)
        x_peer = (1 - my_x, my_y)

        row0 = my_x * HALF_M
        peer_row0 = (1 - my_x) * HALF_M

        loc = pltpu.make_async_copy(
            x_hbm.at[0, pl.ds(row0, HALF_M), pl.ds(my_y * N_COLS, N_COLS)],
            xloc,
            loc_sem,
        )
        loc.start()

        barrier_sem = pltpu.get_barrier_semaphore()
        for nbr in (y_peer, x_peer):
            pl.semaphore_signal(
                barrier_sem, inc=1, device_id=nbr,
                device_id_type=pl.DeviceIdType.MESH,
            )
        pl.semaphore_wait(barrier_sem, 2)

        def fwd_rdma(c):
            return pltpu.make_async_remote_copy(
                src_ref=xloc.at[OFFS[c]:OFFS[c + 1], :],
                dst_ref=out_hbm.at[pl.ds(row0 + OFFS[c], CHUNK_ROWS[c]), :],
                send_sem=x_send.at[c],
                recv_sem=x_recv.at[c],
                device_id=x_peer,
                device_id_type=pl.DeviceIdType.MESH,
            )

        y_rdmas = []
        if _DO_Y:
            for c in range(N_CHUNK):
                rdma = pltpu.make_async_remote_copy(
                    src_ref=x_hbm.at[
                        0,
                        pl.ds(row0 + OFFS[c], CHUNK_ROWS[c]),
                        pl.ds((1 - my_y) * N_COLS, N_COLS),
                    ],
                    dst_ref=pbuf.at[OFFS[c]:OFFS[c + 1], :],
                    send_sem=y_send.at[c],
                    recv_sem=y_recv.at[c],
                    device_id=y_peer,
                    device_id_type=pl.DeviceIdType.MESH,
                )
                rdma.start()
                y_rdmas.append(rdma)

        x_rdmas = []
        if _DO_X and _X_EAGER:
            for c in range(N_CHUNK):
                rdma = fwd_rdma(c)
                rdma.start()
                x_rdmas.append(rdma)

        loc.wait()

        out_dmas = []
        group_of_last_chunk = {g[-1]: gi for gi, g in enumerate(STORE_GROUPS)}
        for c in range(N_CHUNK):
            lo, hi = OFFS[c], OFFS[c + 1]
            if _DO_Y:
                y_rdmas[c].wait_recv()
                xloc[lo:hi, :] = xloc[lo:hi, :] + pbuf[lo:hi, :]
            if _DO_X and not _X_EAGER:
                rdma = fwd_rdma(c)
                rdma.start()
                x_rdmas.append(rdma)
            gi = group_of_last_chunk.get(c)
            if gi is not None:
                glo = OFFS[STORE_GROUPS[gi][0]]
                ghi = OFFS[STORE_GROUPS[gi][-1] + 1]
                dma = pltpu.make_async_copy(
                    xloc.at[glo:ghi, :],
                    out_hbm.at[pl.ds(row0 + glo, ghi - glo), :],
                    out_sems.at[gi],
                )
                dma.start()
                out_dmas.append(dma)

        if _DO_X:
            for c in range(N_CHUNK):
                recv = pltpu.make_async_remote_copy(
                    src_ref=xloc.at[OFFS[c]:OFFS[c + 1], :],
                    dst_ref=out_hbm.at[
                        pl.ds(peer_row0 + OFFS[c], CHUNK_ROWS[c]), :
                    ],
                    send_sem=x_send.at[c],
                    recv_sem=x_recv.at[c],
                    device_id=x_peer,
                    device_id_type=pl.DeviceIdType.MESH,
                )
                recv.wait_recv()

        for dma in out_dmas:
            dma.wait()
        for rdma in y_rdmas + x_rdmas:
            rdma.wait_send()

    return pl.pallas_call(
        body,
        out_shape=jax.ShapeDtypeStruct((m, N_COLS), jnp.float32),
        in_specs=[pl.BlockSpec(memory_space=pltpu.MemorySpace.HBM)],
        out_specs=pl.BlockSpec(memory_space=pltpu.MemorySpace.HBM),
        scratch_shapes=[
            pltpu.VMEM((HALF_M, N_COLS), jnp.float32),
            pltpu.VMEM((HALF_M, N_COLS), jnp.float32),
            pltpu.SemaphoreType.DMA((N_CHUNK,)),
            pltpu.SemaphoreType.DMA((N_CHUNK,)),
            pltpu.SemaphoreType.DMA((N_CHUNK,)),
            pltpu.SemaphoreType.DMA((N_CHUNK,)),
            pltpu.SemaphoreType.DMA,
            pltpu.SemaphoreType.DMA((N_GROUP,)),
        ],
        compiler_params=pltpu.CompilerParams(collective_id=0),
    )(x)
